# baseline (speedup 1.0000x reference)
import functools

import jax
import jax.numpy as jnp
from jax import lax
from jax.experimental import pallas as pl
from jax.experimental.pallas import tpu as pltpu
from jax.experimental.pallas import tpu_sc as plsc

B = 16384
D = 32
NUM_TABLES = 26
NUM = 13
NUM_PAD = 16
H = 256
XW = 896
NG = XW // 128
CHUNK = 32
NCHUNK = 16

_info = plsc.get_sparse_core_info()
NC = _info.num_cores
NS = _info.num_subcores
NW = NC * NS
BPW = B // NW

_COL = [32 + 32 * i for i in range(NUM_TABLES)]
_GROUPS = [[i for i in range(NUM_TABLES) if _COL[i] // 128 == g]
           for g in range(NG)]
NIDX = NUM_TABLES * NCHUNK


def _make_sc_body(groups, with_num):
    gids = list(groups)
    tids = [i for g in gids for i in _GROUPS[g]]
    col0 = 128 * gids[0]

    def body(zeros_hbm, num_hbm, idx_hbm, *rest):
        tables = dict(zip(tids, rest[:len(tids)]))
        x_out = rest[len(tids)]
        idx_all, staging, ring, nfat = rest[len(tids) + 1: len(tids) + 5]
        sems, sem_n, sem_w = rest[len(tids) + 5:]

        wid = lax.axis_index("s") * NC + lax.axis_index("c")
        base = wid * BPW

        pltpu.sync_copy(zeros_hbm, staging)
        pltpu.sync_copy(idx_hbm.at[wid], idx_all)

        def fire(i, slot, c):
            table = tables[i]
            row_id = NCHUNK * i + c

            def blk(k, _):
                idxv = idx_all[row_id, pl.ds(16 * k, 16)]
                startv = lax.shift_left(
                    lax.shift_right_logical(idxv, 3), 3)
                for j in range(16):
                    start = pl.multiple_of(startv[j], 8)
                    pltpu.async_copy(table.at[pl.ds(start, 8)],
                                     ring.at[slot, 16 * k + j],
                                     sems.at[slot])
                return _

            lax.fori_loop(0, CHUNK // 16, blk, None)

        def drain_extract(i, slot, c, p):
            table = tables[i]
            col = _COL[i] % 128
            row_id = NCHUNK * i + c

            def dwait(r, _):
                pltpu.make_async_copy(table.at[pl.ds(0, 8)],
                                      ring.at[slot, r],
                                      sems.at[slot]).wait()
                return _

            lax.fori_loop(0, CHUNK, dwait, None, unroll=4)

            def blk(k, _):
                idxv = idx_all[row_id, pl.ds(16 * k, 16)]
                subv = lax.bitwise_and(idxv, 7)
                for j in range(16):
                    r = 16 * k + j
                    sub = subv[j]
                    for h in range(D // 16):
                        staging[p, r, pl.ds(col + 16 * h, 16)] = (
                            ring[slot, r, sub, pl.ds(16 * h, 16)])
                return _

            lax.fori_loop(0, CHUNK // 16, blk, None)

        for g in gids:
            grp = _GROUPS[g]

            def chunk_body(c, _, g=g, grp=grp):
                p = lax.rem(c, 2)
                for ti in range(min(2, len(grp))):
                    fire(grp[ti], ti % 2, c)
                if with_num and g == 0:
                    noff = pl.multiple_of(base + CHUNK * c, CHUNK)
                    pltpu.async_copy(num_hbm.at[pl.ds(noff, CHUNK)],
                                     nfat, sem_n)

                @pl.when(c >= 2)
                def _wait_prev():
                    pltpu.make_async_copy(
                        staging.at[p],
                        x_out.at[pl.ds(base, CHUNK),
                                 pl.ds(128 * g - col0, 128)],
                        sem_w.at[p]).wait()

                for ti, i in enumerate(grp):
                    drain_extract(i, ti % 2, c, p)
                    if ti + 2 < len(grp):
                        fire(grp[ti + 2], (ti + 2) % 2, c)
                if with_num and g == 0:
                    pltpu.make_async_copy(
                        num_hbm.at[pl.ds(base, CHUNK)], nfat, sem_n).wait()

                    def nrow(r, _):
                        staging[p, r, pl.ds(0, NUM_PAD)] = (
                            nfat[r, pl.ds(0, NUM_PAD)])
                        return _

                    lax.fori_loop(0, CHUNK, nrow, None, unroll=4)

                xoff = pl.multiple_of(base + CHUNK * c, CHUNK)
                pltpu.async_copy(
                    staging.at[p],
                    x_out.at[pl.ds(xoff, CHUNK),
                             pl.ds(128 * g - col0, 128)],
                    sem_w.at[p])
                return _

            lax.fori_loop(0, NCHUNK, chunk_body, None)
            for p in range(2):
                pltpu.make_async_copy(
                    staging.at[p],
                    x_out.at[pl.ds(base, CHUNK), pl.ds(128 * g - col0, 128)],
                    sem_w.at[p]).wait()

    return body, tids, len(gids) * 128


def _make_sc(groups, with_num):
    body, tids, width = _make_sc_body(groups, with_num)
    fn = functools.partial(
        pl.kernel,
        mesh=plsc.VectorSubcoreMesh(core_axis_name="c", subcore_axis_name="s"),
        out_type=jax.ShapeDtypeStruct((B, width), jnp.float32),
        scratch_types=(
            [pltpu.VMEM((NIDX, CHUNK), jnp.int32),
             pltpu.VMEM((2, CHUNK, 128), jnp.float32),
             pltpu.VMEM((2, CHUNK, 8, D), jnp.float32),
             pltpu.VMEM((CHUNK, NUM_PAD), jnp.float32),
             pltpu.SemaphoreType.DMA((2,)),
             pltpu.SemaphoreType.DMA,
             pltpu.SemaphoreType.DMA((2,))]
        ),
    )(body)
    return fn, tids


_sc_a, _tids_a = _make_sc(range(0, 4), True)
_sc_b, _tids_b = _make_sc(range(4, NG), False)

WA = 512
WB = 384
TB = 1024


def _mm_body(xa_ref, xb_ref, w_ref, b_ref, o_ref):
    x = jnp.concatenate([xa_ref[...], xb_ref[...]], axis=1)
    o_ref[...] = (
        jnp.dot(x, w_ref[...], preferred_element_type=jnp.float32)
        + b_ref[...]
    )


_tc_matmul = pl.pallas_call(
    _mm_body,
    grid=(B // TB,),
    in_specs=[
        pl.BlockSpec((TB, WA), lambda i: (i, 0)),
        pl.BlockSpec((TB, WB), lambda i: (i, 0)),
        pl.BlockSpec((XW, H), lambda i: (0, 0)),
        pl.BlockSpec((1, H), lambda i: (0, 0)),
    ],
    out_specs=pl.BlockSpec((TB, H), lambda i: (i, 0)),
    out_shape=jax.ShapeDtypeStruct((B, H), jnp.float32),
)


def kernel(numerical, cat_0, cat_1, cat_2, cat_3, cat_4, cat_5, cat_6, cat_7, cat_8, cat_9, cat_10, cat_11, cat_12, cat_13, cat_14, cat_15, cat_16, cat_17, cat_18, cat_19, cat_20, cat_21, cat_22, cat_23, cat_24, cat_25, emb_0, emb_1, emb_2, emb_3, emb_4, emb_5, emb_6, emb_7, emb_8, emb_9, emb_10, emb_11, emb_12, emb_13, emb_14, emb_15, emb_16, emb_17, emb_18, emb_19, emb_20, emb_21, emb_22, emb_23, emb_24, emb_25, W, b):
    embs = [emb_0, emb_1, emb_2, emb_3, emb_4, emb_5, emb_6, emb_7, emb_8,
            emb_9, emb_10, emb_11, emb_12, emb_13, emb_14, emb_15, emb_16,
            emb_17, emb_18, emb_19, emb_20, emb_21, emb_22, emb_23, emb_24,
            emb_25]
    cats = jnp.stack(
        [cat_0, cat_1, cat_2, cat_3, cat_4, cat_5, cat_6, cat_7, cat_8,
         cat_9, cat_10, cat_11, cat_12, cat_13, cat_14, cat_15, cat_16,
         cat_17, cat_18, cat_19, cat_20, cat_21, cat_22, cat_23, cat_24,
         cat_25], axis=0).astype(jnp.int32)
    idx = cats.reshape(NUM_TABLES, NW, NCHUNK, CHUNK).transpose(
        1, 0, 2, 3).reshape(NW, NIDX, CHUNK)
    num_pad = jnp.pad(numerical, ((0, 0), (0, NUM_PAD - NUM)))
    zeros = jnp.zeros((2, CHUNK, 128), jnp.float32)
    xa = _sc_a(zeros, num_pad, idx, *[embs[i] for i in _tids_a])
    xb = _sc_b(zeros, num_pad, idx, *[embs[i] for i in _tids_b])
    W_pad = jnp.concatenate(
        [W[:NUM], jnp.zeros((32 - NUM, H), W.dtype), W[NUM:],
         jnp.zeros((XW - 32 - NUM_TABLES * D, H), W.dtype)], axis=0)
    return _tc_matmul(xa, xb, W_pad, b.reshape(1, H))


# speedup vs baseline: 1.0116x; 1.0116x over previous
import functools

import jax
import jax.numpy as jnp
from jax import lax
from jax.experimental import pallas as pl
from jax.experimental.pallas import tpu as pltpu
from jax.experimental.pallas import tpu_sc as plsc

B = 16384
D = 32
NUM_TABLES = 26
NUM = 13
NUM_PAD = 16
H = 256
XW = 896
NG = XW // 128
CHUNK = 64
NCHUNK = 8

_info = plsc.get_sparse_core_info()
NC = _info.num_cores
NS = _info.num_subcores
NW = NC * NS
BPW = B // NW

_COL = [32 + 32 * i for i in range(NUM_TABLES)]
_GROUPS = [[i for i in range(NUM_TABLES) if _COL[i] // 128 == g]
           for g in range(NG)]
NIDX = NUM_TABLES * NCHUNK


def _sc_body(zeros_hbm, num_hbm, idxr_hbm, idxo_hbm, *rest):
    tables = rest[:NUM_TABLES]
    x_out = rest[NUM_TABLES]
    idxr, idxo, staging, rows, nfat = rest[NUM_TABLES + 1: NUM_TABLES + 6]
    sems, sem_n, sem_w = rest[NUM_TABLES + 6:]

    wid = lax.axis_index("s") * NC + lax.axis_index("c")
    base = wid * BPW

    pltpu.sync_copy(zeros_hbm, staging)
    pltpu.sync_copy(idxr_hbm.at[wid], idxr)
    pltpu.sync_copy(idxo_hbm.at[wid], idxo)

    def fire(i, slot, c):
        pltpu.async_copy(tables[i].at[idxr.at[NCHUNK * i + c]],
                         rows.at[slot], sems.at[slot])

    def drain_extract(i, slot, c, p):
        col = _COL[i] % 128
        row_id = NCHUNK * i + c
        pltpu.make_async_copy(tables[i].at[idxr.at[row_id]],
                              rows.at[slot], sems.at[slot]).wait()

        def blk(k, _):
            idxv = idxo[row_id, pl.ds(16 * k, 16)]
            offv = lax.shift_left(lax.bitwise_and(idxv, 3), 5)
            for j in range(16):
                r = 16 * k + j
                off = offv[j]
                for h in range(D // 16):
                    staging[p, r, pl.ds(col + 16 * h, 16)] = (
                        rows[slot, r, pl.ds(off + 16 * h, 16)])
            return _

        lax.fori_loop(0, CHUNK // 16, blk, None)

    for g in range(NG):
        grp = _GROUPS[g]

        def chunk_body(c, _, g=g, grp=grp):
            p = lax.rem(c, 2)
            for ti in range(min(2, len(grp))):
                fire(grp[ti], ti % 2, c)
            if g == 0:
                noff = pl.multiple_of((base + CHUNK * c) // 8, 8)
                pltpu.async_copy(num_hbm.at[pl.ds(noff, CHUNK // 8)],
                                 nfat, sem_n)

            @pl.when(c >= 2)
            def _wait_prev():
                pltpu.make_async_copy(
                    staging.at[p],
                    x_out.at[pl.ds(base, CHUNK), pl.ds(128 * g, 128)],
                    sem_w.at[p]).wait()

            for ti, i in enumerate(grp):
                drain_extract(i, ti % 2, c, p)
                if ti + 2 < len(grp):
                    fire(grp[ti + 2], (ti + 2) % 2, c)
            if g == 0:
                pltpu.make_async_copy(
                    num_hbm.at[pl.ds(0, CHUNK // 8)], nfat, sem_n).wait()

                def nblk(k, _):
                    for j in range(16):
                        r = 16 * k + j
                        staging[p, r, pl.ds(0, NUM_PAD)] = (
                            nfat[r // 8, pl.ds(NUM_PAD * (r % 8), NUM_PAD)])
                    return _

                lax.fori_loop(0, CHUNK // 16, nblk, None)

            xoff = pl.multiple_of(base + CHUNK * c, CHUNK)
            pltpu.async_copy(
                staging.at[p],
                x_out.at[pl.ds(xoff, CHUNK), pl.ds(128 * g, 128)],
                sem_w.at[p])
            return _

        lax.fori_loop(0, NCHUNK, chunk_body, None)
        for p in range(2):
            pltpu.make_async_copy(
                staging.at[p],
                x_out.at[pl.ds(base, CHUNK), pl.ds(128 * g, 128)],
                sem_w.at[p]).wait()


_sc_fill = functools.partial(
    pl.kernel,
    mesh=plsc.VectorSubcoreMesh(core_axis_name="c", subcore_axis_name="s"),
    out_type=jax.ShapeDtypeStruct((B, XW), jnp.float32),
    scratch_types=(
        [pltpu.VMEM((NIDX, CHUNK), jnp.int32),
         pltpu.VMEM((NIDX, CHUNK), jnp.int32),
         pltpu.VMEM((2, CHUNK, 128), jnp.float32),
         pltpu.VMEM((2, CHUNK, 128), jnp.float32),
         pltpu.VMEM((CHUNK // 8, 128), jnp.float32),
         pltpu.SemaphoreType.DMA((2,)),
         pltpu.SemaphoreType.DMA,
         pltpu.SemaphoreType.DMA((2,))]
    ),
)(_sc_body)


TB = 1024


def _mm_body(x_ref, w_ref, b_ref, o_ref):
    o_ref[...] = (
        jnp.dot(x_ref[...], w_ref[...], preferred_element_type=jnp.float32)
        + b_ref[...]
    )


_tc_matmul = pl.pallas_call(
    _mm_body,
    grid=(B // TB,),
    in_specs=[
        pl.BlockSpec((TB, XW), lambda i: (i, 0)),
        pl.BlockSpec((XW, H), lambda i: (0, 0)),
        pl.BlockSpec((1, H), lambda i: (0, 0)),
    ],
    out_specs=pl.BlockSpec((TB, H), lambda i: (i, 0)),
    out_shape=jax.ShapeDtypeStruct((B, H), jnp.float32),
)


def kernel(numerical, cat_0, cat_1, cat_2, cat_3, cat_4, cat_5, cat_6, cat_7, cat_8, cat_9, cat_10, cat_11, cat_12, cat_13, cat_14, cat_15, cat_16, cat_17, cat_18, cat_19, cat_20, cat_21, cat_22, cat_23, cat_24, cat_25, emb_0, emb_1, emb_2, emb_3, emb_4, emb_5, emb_6, emb_7, emb_8, emb_9, emb_10, emb_11, emb_12, emb_13, emb_14, emb_15, emb_16, emb_17, emb_18, emb_19, emb_20, emb_21, emb_22, emb_23, emb_24, emb_25, W, b):
    embs = [emb_0, emb_1, emb_2, emb_3, emb_4, emb_5, emb_6, emb_7, emb_8,
            emb_9, emb_10, emb_11, emb_12, emb_13, emb_14, emb_15, emb_16,
            emb_17, emb_18, emb_19, emb_20, emb_21, emb_22, emb_23, emb_24,
            emb_25]
    wide = [e.reshape(e.shape[0] // 4, 128) for e in embs]
    cats = jnp.stack(
        [cat_0, cat_1, cat_2, cat_3, cat_4, cat_5, cat_6, cat_7, cat_8,
         cat_9, cat_10, cat_11, cat_12, cat_13, cat_14, cat_15, cat_16,
         cat_17, cat_18, cat_19, cat_20, cat_21, cat_22, cat_23, cat_24,
         cat_25], axis=0).astype(jnp.int32)
    idx = cats.reshape(NUM_TABLES, NW, NCHUNK, CHUNK).transpose(
        1, 0, 2, 3).reshape(NW, NIDX, CHUNK)
    idx_rows = lax.shift_right_logical(idx, 2)
    num_wide = jnp.pad(
        numerical, ((0, 0), (0, NUM_PAD - NUM))).reshape(B // 8, 128)
    zeros = jnp.zeros((2, CHUNK, 128), jnp.float32)
    X = _sc_fill(zeros, num_wide, idx_rows, idx, *wide)
    W_pad = jnp.concatenate(
        [W[:NUM], jnp.zeros((32 - NUM, H), W.dtype), W[NUM:],
         jnp.zeros((XW - 32 - NUM_TABLES * D, H), W.dtype)], axis=0)
    return _tc_matmul(X, W_pad, b.reshape(1, H))
